# Initial kernel scaffold; baseline (speedup 1.0000x reference)
import jax
import jax.numpy as jnp
from jax.experimental import pallas as pl


def kernel(out, key, idx, key_store, value_store, idx_store, Wp, bp, Wg, a_att, gat_attW, gat_attb, h1, h2, s1, s2):
    raise NotImplementedError("write your pallas kernel here")


# speedup vs baseline: 6.2092x; 6.2092x over previous
import functools
import numpy as np
import jax
import jax.numpy as jnp
from jax import lax
from jax.experimental import pallas as pl
from jax.experimental.pallas import tpu as pltpu
from jax.experimental.pallas import tpu_sc as plsc

SLOTS = 100000
B = 1024
KD = 128
VD = 128
RK = 16
C = 128
SP = 102400
NCH = SP // C
SBLK = 2048
NSTEPS = SP // SBLK
CH_STEP = SBLK // C
BBLK = 256
NEG = -1e30
NEG2 = -3e38
IBIG = 2 ** 30

NCORES = 2
NSUB = 16
NW = NCORES * NSUB


def _sim_body(key_ref, wp_ref, bp_ref, ks_ref, sim_ref, m_ref, pkn_ref):
    i = pl.program_id(0)

    @pl.when(i == 0)
    def _():
        k = key_ref[...]
        pk = lax.dot_general(k, wp_ref[...], (((1,), (1,)), ((), ())),
                             preferred_element_type=jnp.float32) + bp_ref[...]
        n = jnp.maximum(jnp.sqrt(jnp.sum(pk * pk, axis=1, keepdims=True)), 1e-8)
        pkn_ref[...] = pk / n

    ks = ks_ref[...]
    pm = lax.dot_general(ks, wp_ref[...], (((1,), (1,)), ((), ())),
                         preferred_element_type=jnp.float32) + bp_ref[...]
    n = jnp.maximum(jnp.sqrt(jnp.sum(pm * pm, axis=1, keepdims=True)), 1e-8)
    pmn = pm / n
    simb = lax.dot_general(pkn_ref[...], pmn, (((1,), (1,)), ((), ())),
                           preferred_element_type=jnp.float32)
    col = lax.broadcasted_iota(jnp.int32, (B, SBLK), 1)
    simb = jnp.where(col + i * SBLK < SLOTS, simb, NEG)
    sim_ref[...] = simb
    parts = [jnp.max(simb[:, c * C:(c + 1) * C], axis=1, keepdims=True)
             for c in range(CH_STEP)]
    m_ref[...] = jnp.concatenate(parts, axis=1)[None]


def _stage_a(qkey, Wp, bp2, ksp):
    return pl.pallas_call(
        _sim_body,
        grid=(NSTEPS,),
        in_specs=[
            pl.BlockSpec((B, KD), lambda i: (0, 0)),
            pl.BlockSpec((KD, KD), lambda i: (0, 0)),
            pl.BlockSpec((1, KD), lambda i: (0, 0)),
            pl.BlockSpec((SBLK, KD), lambda i: (i, 0)),
        ],
        out_specs=[
            pl.BlockSpec((B, SBLK), lambda i: (0, i)),
            pl.BlockSpec((1, B, CH_STEP), lambda i: (i, 0, 0)),
        ],
        out_shape=[
            jax.ShapeDtypeStruct((B, SP), jnp.float32),
            jax.ShapeDtypeStruct((NSTEPS, B, CH_STEP), jnp.float32),
        ],
        scratch_shapes=[pltpu.VMEM((B, KD), jnp.float32)],
    )(qkey, Wp, bp2, ksp)


def _extract_body(m_ref, ids_ref):
    x = m_ref[...]
    col = lax.broadcasted_iota(jnp.int32, (BBLK, NCH), 1)
    outs = []
    for _ in range(RK):
        mx = jnp.max(x, axis=1, keepdims=True)
        ci = jnp.where(x >= mx, col, IBIG)
        c = jnp.min(ci, axis=1, keepdims=True)
        outs.append(c)
        x = jnp.where(col == c, NEG2, x)
    ids_ref[...] = jnp.concatenate(outs, axis=1)


def _stage_b(M):
    return pl.pallas_call(
        _extract_body,
        grid=(B // BBLK,),
        in_specs=[pl.BlockSpec((BBLK, NCH), lambda i: (i, 0))],
        out_specs=pl.BlockSpec((BBLK, RK), lambda i: (i, 0)),
        out_shape=jax.ShapeDtypeStruct((B, RK), jnp.int32),
    )(M)


def _gather_rows(table, idx):
    nb = idx.shape[0]
    bpw = nb // NW
    nstream = bpw // 128
    idx2 = idx.reshape(nb // 128, 128)
    mesh = plsc.VectorSubcoreMesh(core_axis_name="c", subcore_axis_name="s",
                                  num_cores=NCORES, num_subcores=NSUB)

    @functools.partial(
        pl.kernel, mesh=mesh,
        out_type=jax.ShapeDtypeStruct((nb, VD), jnp.float32),
        scratch_types=[
            pltpu.VMEM((nstream, 128), jnp.int32),
            pltpu.VMEM((bpw, VD), jnp.float32),
            pltpu.SemaphoreType.DMA,
        ],
    )
    def k(table_hbm, idx_hbm, out_hbm, idx_v, rows_v, sem):
        wid = lax.axis_index("s") * NCORES + lax.axis_index("c")
        pltpu.sync_copy(idx_hbm.at[pl.ds(wid * nstream, nstream)], idx_v)
        copies = []
        for j in range(nstream):
            copies.append(pltpu.async_copy(
                table_hbm.at[idx_v.at[j]],
                rows_v.at[pl.ds(j * 128, 128)], sem))
        for cp in copies:
            cp.wait()
        pltpu.sync_copy(rows_v, out_hbm.at[pl.ds(wid * bpw, bpw)])

    return k(table, idx2)


def _rescan_body(cand_ref, cid_ref, ids_ref):
    x = cand_ref[...]
    cid = cid_ref[...]
    o = lax.broadcasted_iota(jnp.int32, (BBLK, C), 1)
    slotid = jnp.concatenate([cid[:, j:j + 1] * C + o for j in range(RK)], axis=1)
    outs = []
    for _ in range(RK):
        mx = jnp.max(x, axis=1, keepdims=True)
        ci = jnp.where(x >= mx, slotid, IBIG)
        s = jnp.min(ci, axis=1, keepdims=True)
        outs.append(s)
        x = jnp.where(slotid == s, NEG2, x)
    ids_ref[...] = jnp.concatenate(outs, axis=1)


def _stage_c(cand, cids):
    return pl.pallas_call(
        _rescan_body,
        grid=(B // BBLK,),
        in_specs=[
            pl.BlockSpec((BBLK, RK * C), lambda i: (i, 0)),
            pl.BlockSpec((BBLK, RK), lambda i: (i, 0)),
        ],
        out_specs=pl.BlockSpec((BBLK, RK), lambda i: (i, 0)),
        out_shape=jax.ShapeDtypeStruct((B, RK), jnp.int32),
    )(cand, cids)


def _leaky(x):
    return jnp.where(x >= 0, x, 0.2 * x)


def _compose_body(out_ref, sel_ref, wg_ref, a1_ref, a2_ref, e4_ref, gw_ref,
                  fc1_ref, fs1_ref, fc2_ref, fs2_ref, ic_ref, isn_ref, o_ref):
    ob = out_ref[...]
    sel = sel_ref[...]
    Wg = wg_ref[...]
    selF = sel.reshape(BBLK * RK, VD)
    H0 = lax.dot_general(ob, Wg, (((1,), (0,)), ((), ())),
                         preferred_element_type=jnp.float32)
    Hm = lax.dot_general(selF, Wg, (((1,), (0,)), ((), ())),
                         preferred_element_type=jnp.float32)

    a1 = a1_ref[...]
    a2 = a2_ref[...]
    es0 = lax.dot_general(H0, a1, (((1,), (0,)), ((), ())),
                          preferred_element_type=jnp.float32)
    ed0 = lax.dot_general(H0, a2, (((1,), (0,)), ((), ())),
                          preferred_element_type=jnp.float32)
    edm = lax.dot_general(Hm, a2, (((1,), (0,)), ((), ())),
                          preferred_element_type=jnp.float32)
    edm3 = edm.reshape(BBLK, RK, 4)

    z0 = _leaky(es0 + ed0)
    zm3 = _leaky(es0[:, None, :] + edm3)
    mxz = jnp.maximum(jnp.max(zm3, axis=1), z0)
    w0 = jnp.exp(z0 - mxz)
    wm3 = jnp.exp(zm3 - mxz[:, None, :])
    den = w0 + jnp.sum(wm3, axis=1)
    a0 = w0 / den
    amF = (wm3 / den[:, None, :]).reshape(BBLK * RK, 4)

    e4 = e4_ref[...]
    w0E = lax.dot_general(a0, e4, (((1,), (0,)), ((), ())),
                          preferred_element_type=jnp.float32)
    wmE = lax.dot_general(amF, e4, (((1,), (0,)), ((), ())),
                          preferred_element_type=jnp.float32)
    gat0 = w0E * H0 + jnp.sum((wmE * Hm).reshape(BBLK, RK, VD), axis=1)

    gwr = gw_ref[...]
    Hm3 = Hm.reshape(BBLK, RK, VD)
    s0 = jnp.sum(gat0 * gwr, axis=1, keepdims=True)
    sm = jnp.sum(Hm3 * gwr[:, None, :], axis=2)
    mx2 = jnp.maximum(jnp.max(sm, axis=1, keepdims=True), s0)
    e0 = jnp.exp(s0 - mx2)
    em = jnp.exp(sm - mx2)
    den2 = e0 + jnp.sum(em, axis=1, keepdims=True)
    pooled = (e0 / den2) * gat0 + jnp.sum(
        (em / den2)[:, :, None] * Hm3, axis=1)

    f1r = lax.dot_general(ob, fc1_ref[...], (((1,), (0,)), ((), ())),
                          preferred_element_type=jnp.float32)
    f1i = lax.dot_general(ob, fs1_ref[...], (((1,), (0,)), ((), ())),
                          preferred_element_type=jnp.float32)
    f2r = lax.dot_general(pooled, fc2_ref[...], (((1,), (0,)), ((), ())),
                          preferred_element_type=jnp.float32)
    f2i = lax.dot_general(pooled, fs2_ref[...], (((1,), (0,)), ((), ())),
                          preferred_element_type=jnp.float32)
    pr = f1r * f2r - f1i * f2i
    pi = f1r * f2i + f1i * f2r
    cb = (lax.dot_general(pr, ic_ref[...], (((1,), (0,)), ((), ())),
                          preferred_element_type=jnp.float32)
          - lax.dot_general(pi, isn_ref[...], (((1,), (0,)), ((), ())),
                            preferred_element_type=jnp.float32))

    o_ref[:, :VD] = cb
    o_ref[:, VD:] = ob


def _stage_d(out, sel3, Wg, A1, A2, E4, gwr, FC1, FS1, FC2, FS2, IC, ISn):
    full = lambda shape: pl.BlockSpec(shape, lambda i: tuple(0 for _ in shape))
    return pl.pallas_call(
        _compose_body,
        grid=(B // BBLK,),
        in_specs=[
            pl.BlockSpec((BBLK, VD), lambda i: (i, 0)),
            pl.BlockSpec((BBLK, RK, VD), lambda i: (i, 0, 0)),
            full((VD, VD)),
            full((VD, 4)),
            full((VD, 4)),
            full((4, VD)),
            full((1, VD)),
            full((VD, VD)),
            full((VD, VD)),
            full((VD, VD)),
            full((VD, VD)),
            full((VD, VD)),
            full((VD, VD)),
        ],
        out_specs=pl.BlockSpec((BBLK, 2 * VD), lambda i: (i, 0)),
        out_shape=jax.ShapeDtypeStruct((B, 2 * VD), jnp.float32),
    )(out, sel3, Wg, A1, A2, E4, gwr, FC1, FS1, FC2, FS2, IC, ISn)


_n = np.arange(128)
_ang = 2.0 * np.pi * ((np.outer(_n, _n) % 128).astype(np.float64)) / 128.0
_COS = np.cos(_ang)
_SIN = np.sin(_ang)


def kernel(out, key, idx, key_store, value_store, idx_store, Wp, bp, Wg,
           a_att, gat_attW, gat_attb, h1, h2, s1, s2):
    ksp = jnp.pad(key_store, ((0, SP - SLOTS), (0, 0)))
    bp2 = bp.reshape(1, KD)
    dh = VD // 4
    rows = jnp.arange(VD)
    head = rows // dh
    A1 = jnp.zeros((VD, 4), jnp.float32).at[rows, head].set(
        a_att[:, :dh].reshape(VD))
    A2 = jnp.zeros((VD, 4), jnp.float32).at[rows, head].set(
        a_att[:, dh:].reshape(VD))
    E4 = (head[None, :] == jnp.arange(4)[:, None]).astype(jnp.float32)
    gwr = gat_attW.reshape(1, VD)

    cosc = jnp.asarray(_COS, jnp.float32)
    sinc = jnp.asarray(_SIN, jnp.float32)
    FC1 = s1[:, None] * cosc[h1, :]
    FS1 = s1[:, None] * (-sinc)[h1, :]
    FC2 = s2[:, None] * cosc[h2, :]
    FS2 = s2[:, None] * (-sinc)[h2, :]
    IC = jnp.asarray(_COS / 128.0, jnp.float32)
    ISn = jnp.asarray(_SIN / 128.0, jnp.float32)

    sim, M3 = _stage_a(key, Wp, bp2, ksp)
    M = M3.transpose(1, 0, 2).reshape(B, NCH)
    cids = _stage_b(M)
    crow = (jnp.arange(B, dtype=jnp.int32)[:, None] * NCH + cids).reshape(-1)
    cand = _gather_rows(sim.reshape(B * NCH, C), crow)
    fids = _stage_c(cand.reshape(B, RK * C), cids)
    sel = _gather_rows(value_store, fids.reshape(-1))

    return _stage_d(out, sel.reshape(B, RK, VD), Wg, A1, A2, E4, gwr,
                    FC1, FS1, FC2, FS2, IC, ISn)


# speedup vs baseline: 9.4766x; 1.5262x over previous
import functools
import numpy as np
import jax
import jax.numpy as jnp
from jax import lax
from jax.experimental import pallas as pl
from jax.experimental.pallas import tpu as pltpu
from jax.experimental.pallas import tpu_sc as plsc

SLOTS = 100000
B = 1024
KD = 128
VD = 128
RK = 16
C = 128
SP = 102400
NCH = SP // C
SBLK = 2048
NSTEPS = SP // SBLK
CH_STEP = SBLK // C
BBLK = 256
NEG = -1e30
NEG2 = -3e38
IBIG = 2 ** 30

NCORES = 2
NSUB = 16
NW = NCORES * NSUB


def _sim_body(key_ref, wp_ref, bp_ref, ks_ref, sim_ref, m_ref, pkn_ref):
    i = pl.program_id(0)

    @pl.when(i == 0)
    def _():
        k = key_ref[...]
        pk = lax.dot_general(k, wp_ref[...], (((1,), (1,)), ((), ())),
                             preferred_element_type=jnp.float32) + bp_ref[...]
        n = jnp.maximum(jnp.sqrt(jnp.sum(pk * pk, axis=1, keepdims=True)), 1e-8)
        pkn_ref[...] = pk / n

    ks = ks_ref[...]
    pm = lax.dot_general(ks, wp_ref[...], (((1,), (1,)), ((), ())),
                         preferred_element_type=jnp.float32) + bp_ref[...]
    n = jnp.maximum(jnp.sqrt(jnp.sum(pm * pm, axis=1, keepdims=True)), 1e-8)
    pmn = pm / n
    simb = lax.dot_general(pkn_ref[...], pmn, (((1,), (1,)), ((), ())),
                           preferred_element_type=jnp.float32)
    col = lax.broadcasted_iota(jnp.int32, (B, SBLK), 1)
    simb = jnp.where(col + i * SBLK < SLOTS, simb, NEG)
    parts = []
    for c in range(CH_STEP):
        blk = simb[:, c * C:(c + 1) * C]
        sim_ref[c] = blk
        parts.append(jnp.max(blk, axis=1, keepdims=True))
    m_ref[...] = jnp.concatenate(parts, axis=1)[None]


def _stage_a(qkey, Wp, bp2, ksp):
    return pl.pallas_call(
        _sim_body,
        grid=(NSTEPS,),
        in_specs=[
            pl.BlockSpec((B, KD), lambda i: (0, 0)),
            pl.BlockSpec((KD, KD), lambda i: (0, 0)),
            pl.BlockSpec((1, KD), lambda i: (0, 0)),
            pl.BlockSpec((SBLK, KD), lambda i: (i, 0)),
        ],
        out_specs=[
            pl.BlockSpec((CH_STEP, B, C), lambda i: (i, 0, 0)),
            pl.BlockSpec((1, B, CH_STEP), lambda i: (i, 0, 0)),
        ],
        out_shape=[
            jax.ShapeDtypeStruct((NCH, B, C), jnp.float32),
            jax.ShapeDtypeStruct((NSTEPS, B, CH_STEP), jnp.float32),
        ],
        scratch_shapes=[pltpu.VMEM((B, KD), jnp.float32)],
    )(qkey, Wp, bp2, ksp)


def _extract_body(m_ref, ids_ref):
    x = m_ref[...]
    col = lax.broadcasted_iota(jnp.int32, (BBLK, NCH), 1)
    outs = []
    for _ in range(RK):
        mx = jnp.max(x, axis=1, keepdims=True)
        ci = jnp.where(x >= mx, col, IBIG)
        c = jnp.min(ci, axis=1, keepdims=True)
        outs.append(c)
        x = jnp.where(col == c, NEG2, x)
    ids_ref[...] = jnp.concatenate(outs, axis=1)


def _stage_b(M):
    return pl.pallas_call(
        _extract_body,
        grid=(B // BBLK,),
        in_specs=[pl.BlockSpec((BBLK, NCH), lambda i: (i, 0))],
        out_specs=pl.BlockSpec((BBLK, RK), lambda i: (i, 0)),
        out_shape=jax.ShapeDtypeStruct((B, RK), jnp.int32),
    )(M)


def _gather_rows(table, idx):
    nb = idx.shape[0]
    bpw = nb // NW
    nstream = bpw // 128
    idx2 = idx.reshape(nb // 128, 128)
    mesh = plsc.VectorSubcoreMesh(core_axis_name="c", subcore_axis_name="s",
                                  num_cores=NCORES, num_subcores=NSUB)

    @functools.partial(
        pl.kernel, mesh=mesh,
        out_type=jax.ShapeDtypeStruct((nb, VD), jnp.float32),
        scratch_types=[
            pltpu.VMEM((nstream, 128), jnp.int32),
            pltpu.VMEM((bpw, VD), jnp.float32),
            pltpu.SemaphoreType.DMA,
        ],
    )
    def k(table_hbm, idx_hbm, out_hbm, idx_v, rows_v, sem):
        wid = lax.axis_index("s") * NCORES + lax.axis_index("c")
        pltpu.sync_copy(idx_hbm.at[pl.ds(wid * nstream, nstream)], idx_v)
        copies = []
        for j in range(nstream):
            copies.append(pltpu.async_copy(
                table_hbm.at[idx_v.at[j]],
                rows_v.at[pl.ds(j * 128, 128)], sem))
        for cp in copies:
            cp.wait()
        pltpu.sync_copy(rows_v, out_hbm.at[pl.ds(wid * bpw, bpw)])

    return k(table, idx2)


def _rescan_body(cand_ref, cid_ref, ids_ref):
    x = cand_ref[...]
    cid = cid_ref[...]
    o = lax.broadcasted_iota(jnp.int32, (BBLK, RK, C), 2)
    slotid = cid[:, :, None] * C + o
    outs = []
    for _ in range(RK):
        mx = jnp.max(jnp.max(x, axis=2, keepdims=True), axis=1, keepdims=True)
        ci = jnp.where(x >= mx, slotid, IBIG)
        s = jnp.min(jnp.min(ci, axis=2, keepdims=True), axis=1, keepdims=True)
        outs.append(s[:, :, 0])
        x = jnp.where(slotid == s, NEG2, x)
    ids_ref[...] = jnp.concatenate(outs, axis=1)


def _stage_c(cand3, cids):
    return pl.pallas_call(
        _rescan_body,
        grid=(B // BBLK,),
        in_specs=[
            pl.BlockSpec((BBLK, RK, C), lambda i: (i, 0, 0)),
            pl.BlockSpec((BBLK, RK), lambda i: (i, 0)),
        ],
        out_specs=pl.BlockSpec((BBLK, RK), lambda i: (i, 0)),
        out_shape=jax.ShapeDtypeStruct((B, RK), jnp.int32),
    )(cand3, cids)


def _leaky(x):
    return jnp.where(x >= 0, x, 0.2 * x)


def _compose_body(out_ref, sel_ref, wg_ref, a1_ref, a2_ref, e4_ref, gw_ref,
                  fc1_ref, fs1_ref, fc2_ref, fs2_ref, ic_ref, isn_ref, o_ref):
    ob = out_ref[...]
    sel = sel_ref[...]
    Wg = wg_ref[...]
    selF = sel.reshape(BBLK * RK, VD)
    H0 = lax.dot_general(ob, Wg, (((1,), (0,)), ((), ())),
                         preferred_element_type=jnp.float32)
    Hm = lax.dot_general(selF, Wg, (((1,), (0,)), ((), ())),
                         preferred_element_type=jnp.float32)

    a1 = a1_ref[...]
    a2 = a2_ref[...]
    es0 = lax.dot_general(H0, a1, (((1,), (0,)), ((), ())),
                          preferred_element_type=jnp.float32)
    ed0 = lax.dot_general(H0, a2, (((1,), (0,)), ((), ())),
                          preferred_element_type=jnp.float32)
    edm = lax.dot_general(Hm, a2, (((1,), (0,)), ((), ())),
                          preferred_element_type=jnp.float32)
    edm3 = edm.reshape(BBLK, RK, 4)

    z0 = _leaky(es0 + ed0)
    zm3 = _leaky(es0[:, None, :] + edm3)
    mxz = jnp.maximum(jnp.max(zm3, axis=1), z0)
    w0 = jnp.exp(z0 - mxz)
    wm3 = jnp.exp(zm3 - mxz[:, None, :])
    den = w0 + jnp.sum(wm3, axis=1)
    a0 = w0 / den
    amF = (wm3 / den[:, None, :]).reshape(BBLK * RK, 4)

    e4 = e4_ref[...]
    w0E = lax.dot_general(a0, e4, (((1,), (0,)), ((), ())),
                          preferred_element_type=jnp.float32)
    wmE = lax.dot_general(amF, e4, (((1,), (0,)), ((), ())),
                          preferred_element_type=jnp.float32)
    gat0 = w0E * H0 + jnp.sum((wmE * Hm).reshape(BBLK, RK, VD), axis=1)

    gwr = gw_ref[...]
    Hm3 = Hm.reshape(BBLK, RK, VD)
    s0 = jnp.sum(gat0 * gwr, axis=1, keepdims=True)
    sm = jnp.sum(Hm3 * gwr[:, None, :], axis=2)
    mx2 = jnp.maximum(jnp.max(sm, axis=1, keepdims=True), s0)
    e0 = jnp.exp(s0 - mx2)
    em = jnp.exp(sm - mx2)
    den2 = e0 + jnp.sum(em, axis=1, keepdims=True)
    pooled = (e0 / den2) * gat0 + jnp.sum(
        (em / den2)[:, :, None] * Hm3, axis=1)

    f1r = lax.dot_general(ob, fc1_ref[...], (((1,), (0,)), ((), ())),
                          preferred_element_type=jnp.float32)
    f1i = lax.dot_general(ob, fs1_ref[...], (((1,), (0,)), ((), ())),
                          preferred_element_type=jnp.float32)
    f2r = lax.dot_general(pooled, fc2_ref[...], (((1,), (0,)), ((), ())),
                          preferred_element_type=jnp.float32)
    f2i = lax.dot_general(pooled, fs2_ref[...], (((1,), (0,)), ((), ())),
                          preferred_element_type=jnp.float32)
    pr = f1r * f2r - f1i * f2i
    pi = f1r * f2i + f1i * f2r
    cb = (lax.dot_general(pr, ic_ref[...], (((1,), (0,)), ((), ())),
                          preferred_element_type=jnp.float32)
          - lax.dot_general(pi, isn_ref[...], (((1,), (0,)), ((), ())),
                            preferred_element_type=jnp.float32))

    o_ref[:, :VD] = cb
    o_ref[:, VD:] = ob


def _stage_d(out, sel3, Wg, A1, A2, E4, gwr, FC1, FS1, FC2, FS2, IC, ISn):
    full = lambda shape: pl.BlockSpec(shape, lambda i: tuple(0 for _ in shape))
    return pl.pallas_call(
        _compose_body,
        grid=(B // BBLK,),
        in_specs=[
            pl.BlockSpec((BBLK, VD), lambda i: (i, 0)),
            pl.BlockSpec((BBLK, RK, VD), lambda i: (i, 0, 0)),
            full((VD, VD)),
            full((VD, 4)),
            full((VD, 4)),
            full((4, VD)),
            full((1, VD)),
            full((VD, VD)),
            full((VD, VD)),
            full((VD, VD)),
            full((VD, VD)),
            full((VD, VD)),
            full((VD, VD)),
        ],
        out_specs=pl.BlockSpec((BBLK, 2 * VD), lambda i: (i, 0)),
        out_shape=jax.ShapeDtypeStruct((B, 2 * VD), jnp.float32),
    )(out, sel3, Wg, A1, A2, E4, gwr, FC1, FS1, FC2, FS2, IC, ISn)


_n = np.arange(128)
_ang = 2.0 * np.pi * ((np.outer(_n, _n) % 128).astype(np.float64)) / 128.0
_COS = np.cos(_ang)
_SIN = np.sin(_ang)


def kernel(out, key, idx, key_store, value_store, idx_store, Wp, bp, Wg,
           a_att, gat_attW, gat_attb, h1, h2, s1, s2):
    ksp = jnp.pad(key_store, ((0, SP - SLOTS), (0, 0)))
    bp2 = bp.reshape(1, KD)
    dh = VD // 4
    rows = jnp.arange(VD)
    head = rows // dh
    A1 = jnp.zeros((VD, 4), jnp.float32).at[rows, head].set(
        a_att[:, :dh].reshape(VD))
    A2 = jnp.zeros((VD, 4), jnp.float32).at[rows, head].set(
        a_att[:, dh:].reshape(VD))
    E4 = (head[None, :] == jnp.arange(4)[:, None]).astype(jnp.float32)
    gwr = gat_attW.reshape(1, VD)

    cosc = jnp.asarray(_COS, jnp.float32)
    sinc = jnp.asarray(_SIN, jnp.float32)
    FC1 = s1[:, None] * cosc[h1, :]
    FS1 = s1[:, None] * (-sinc)[h1, :]
    FC2 = s2[:, None] * cosc[h2, :]
    FS2 = s2[:, None] * (-sinc)[h2, :]
    IC = jnp.asarray(_COS / 128.0, jnp.float32)
    ISn = jnp.asarray(_SIN / 128.0, jnp.float32)

    sim3, M3 = _stage_a(key, Wp, bp2, ksp)
    M = M3.transpose(1, 0, 2).reshape(B, NCH)
    cids = _stage_b(M)
    crow = (cids * B + jnp.arange(B, dtype=jnp.int32)[:, None]).reshape(-1)
    cand = _gather_rows(sim3.reshape(NCH * B, C), crow)
    fids = _stage_c(cand.reshape(B, RK, C), cids)
    sel = _gather_rows(value_store, fids.reshape(-1))

    return _stage_d(out, sel.reshape(B, RK, VD), Wg, A1, A2, E4, gwr,
                    FC1, FS1, FC2, FS2, IC, ISn)


# speedup vs baseline: 9.8389x; 1.0382x over previous
import functools
import numpy as np
import jax
import jax.numpy as jnp
from jax import lax
from jax.experimental import pallas as pl
from jax.experimental.pallas import tpu as pltpu
from jax.experimental.pallas import tpu_sc as plsc

SLOTS = 100000
B = 1024
KD = 128
VD = 128
RK = 16
C = 128
SP = 102400
NCH = SP // C
SBLK = 4096
NSTEPS = SP // SBLK
CH_STEP = SBLK // C
BBLK = 256
NEG = -1e30
NEG2 = -3e38
IBIG = 2 ** 30

NCORES = 2
NSUB = 16
NW = NCORES * NSUB


def _sim_body(key_ref, wp_ref, bp_ref, ks_ref, sim_ref, m_ref, pkn_ref):
    i = pl.program_id(0)

    @pl.when(i == 0)
    def _():
        k = key_ref[...]
        pk = lax.dot_general(k, wp_ref[...], (((1,), (1,)), ((), ())),
                             preferred_element_type=jnp.float32) + bp_ref[...]
        n = jnp.maximum(jnp.sqrt(jnp.sum(pk * pk, axis=1, keepdims=True)), 1e-8)
        pkn_ref[...] = pk / n

    ks = ks_ref[...]
    pm = lax.dot_general(ks, wp_ref[...], (((1,), (1,)), ((), ())),
                         preferred_element_type=jnp.float32) + bp_ref[...]
    n = jnp.maximum(jnp.sqrt(jnp.sum(pm * pm, axis=1, keepdims=True)), 1e-8)
    pmn = pm / n
    simb = lax.dot_general(pkn_ref[...], pmn, (((1,), (1,)), ((), ())),
                           preferred_element_type=jnp.float32)
    col = lax.broadcasted_iota(jnp.int32, (B, SBLK), 1)
    simb = jnp.where(col + i * SBLK < SLOTS, simb, NEG)
    parts = []
    for c in range(CH_STEP):
        blk = simb[:, c * C:(c + 1) * C]
        sim_ref[c] = blk
        parts.append(jnp.max(blk, axis=1, keepdims=True))
    m_ref[...] = jnp.concatenate(parts, axis=1)[None]


def _stage_a(qkey, Wp, bp2, ksp):
    return pl.pallas_call(
        _sim_body,
        grid=(NSTEPS,),
        in_specs=[
            pl.BlockSpec((B, KD), lambda i: (0, 0)),
            pl.BlockSpec((KD, KD), lambda i: (0, 0)),
            pl.BlockSpec((1, KD), lambda i: (0, 0)),
            pl.BlockSpec((SBLK, KD), lambda i: (i, 0)),
        ],
        out_specs=[
            pl.BlockSpec((CH_STEP, B, C), lambda i: (i, 0, 0)),
            pl.BlockSpec((1, B, CH_STEP), lambda i: (i, 0, 0)),
        ],
        out_shape=[
            jax.ShapeDtypeStruct((NCH, B, C), jnp.float32),
            jax.ShapeDtypeStruct((NSTEPS, B, CH_STEP), jnp.float32),
        ],
        scratch_shapes=[pltpu.VMEM((B, KD), jnp.float32)],
    )(qkey, Wp, bp2, ksp)


def _extract_body(m_ref, ids_ref):
    x = m_ref[...]
    col = lax.broadcasted_iota(jnp.int32, (BBLK, NCH), 1)
    outs = []
    for _ in range(RK):
        mx = jnp.max(x, axis=1, keepdims=True)
        ci = jnp.where(x >= mx, col, IBIG)
        c = jnp.min(ci, axis=1, keepdims=True)
        outs.append(c)
        x = jnp.where(col == c, NEG2, x)
    ids_ref[...] = jnp.concatenate(outs, axis=1)


def _stage_b(M):
    return pl.pallas_call(
        _extract_body,
        grid=(B // BBLK,),
        in_specs=[pl.BlockSpec((BBLK, NCH), lambda i: (i, 0))],
        out_specs=pl.BlockSpec((BBLK, RK), lambda i: (i, 0)),
        out_shape=jax.ShapeDtypeStruct((B, RK), jnp.int32),
    )(M)


def _gather_rows(table, idx):
    nb = idx.shape[0]
    bpw = nb // NW
    nstream = bpw // 128
    idx2 = idx.reshape(nb // 128, 128)
    mesh = plsc.VectorSubcoreMesh(core_axis_name="c", subcore_axis_name="s",
                                  num_cores=NCORES, num_subcores=NSUB)

    @functools.partial(
        pl.kernel, mesh=mesh,
        out_type=jax.ShapeDtypeStruct((nb, VD), jnp.float32),
        scratch_types=[
            pltpu.VMEM((nstream, 128), jnp.int32),
            pltpu.VMEM((bpw, VD), jnp.float32),
            pltpu.SemaphoreType.DMA,
        ],
    )
    def k(table_hbm, idx_hbm, out_hbm, idx_v, rows_v, sem):
        wid = lax.axis_index("s") * NCORES + lax.axis_index("c")
        pltpu.sync_copy(idx_hbm.at[pl.ds(wid * nstream, nstream)], idx_v)
        copies = []
        for j in range(nstream):
            copies.append(pltpu.async_copy(
                table_hbm.at[idx_v.at[j]],
                rows_v.at[pl.ds(j * 128, 128)], sem))
        for cp in copies:
            cp.wait()
        pltpu.sync_copy(rows_v, out_hbm.at[pl.ds(wid * bpw, bpw)])

    return k(table, idx2)


def _rescan_body(cand_ref, cid_ref, ids_ref):
    x = cand_ref[...]
    cid = cid_ref[...]
    o = lax.broadcasted_iota(jnp.int32, (BBLK, RK, C), 2)
    slotid = cid[:, :, None] * C + o
    outs = []
    for _ in range(RK):
        mx = jnp.max(jnp.max(x, axis=2, keepdims=True), axis=1, keepdims=True)
        ci = jnp.where(x >= mx, slotid, IBIG)
        s = jnp.min(jnp.min(ci, axis=2, keepdims=True), axis=1, keepdims=True)
        outs.append(s[:, :, 0])
        x = jnp.where(slotid == s, NEG2, x)
    ids_ref[...] = jnp.concatenate(outs, axis=1)


def _stage_c(cand3, cids):
    return pl.pallas_call(
        _rescan_body,
        grid=(B // BBLK,),
        in_specs=[
            pl.BlockSpec((BBLK, RK, C), lambda i: (i, 0, 0)),
            pl.BlockSpec((BBLK, RK), lambda i: (i, 0)),
        ],
        out_specs=pl.BlockSpec((BBLK, RK), lambda i: (i, 0)),
        out_shape=jax.ShapeDtypeStruct((B, RK), jnp.int32),
    )(cand3, cids)


def _leaky(x):
    return jnp.where(x >= 0, x, 0.2 * x)


def _compose_body(out_ref, sel_ref, wg_ref, a1_ref, a2_ref, e4_ref, gw_ref,
                  fc1_ref, fs1_ref, fc2_ref, fs2_ref, ic_ref, isn_ref, o_ref):
    ob = out_ref[...]
    sel = sel_ref[...]
    Wg = wg_ref[...]
    selF = sel.reshape(BBLK * RK, VD)
    H0 = lax.dot_general(ob, Wg, (((1,), (0,)), ((), ())),
                         preferred_element_type=jnp.float32)
    Hm = lax.dot_general(selF, Wg, (((1,), (0,)), ((), ())),
                         preferred_element_type=jnp.float32)

    a1 = a1_ref[...]
    a2 = a2_ref[...]
    es0 = lax.dot_general(H0, a1, (((1,), (0,)), ((), ())),
                          preferred_element_type=jnp.float32)
    ed0 = lax.dot_general(H0, a2, (((1,), (0,)), ((), ())),
                          preferred_element_type=jnp.float32)
    edm = lax.dot_general(Hm, a2, (((1,), (0,)), ((), ())),
                          preferred_element_type=jnp.float32)
    edm3 = edm.reshape(BBLK, RK, 4)

    z0 = _leaky(es0 + ed0)
    zm3 = _leaky(es0[:, None, :] + edm3)
    mxz = jnp.maximum(jnp.max(zm3, axis=1), z0)
    w0 = jnp.exp(z0 - mxz)
    wm3 = jnp.exp(zm3 - mxz[:, None, :])
    den = w0 + jnp.sum(wm3, axis=1)
    a0 = w0 / den
    amF = (wm3 / den[:, None, :]).reshape(BBLK * RK, 4)

    e4 = e4_ref[...]
    w0E = lax.dot_general(a0, e4, (((1,), (0,)), ((), ())),
                          preferred_element_type=jnp.float32)
    wmE = lax.dot_general(amF, e4, (((1,), (0,)), ((), ())),
                          preferred_element_type=jnp.float32)
    gat0 = w0E * H0 + jnp.sum((wmE * Hm).reshape(BBLK, RK, VD), axis=1)

    gwr = gw_ref[...]
    Hm3 = Hm.reshape(BBLK, RK, VD)
    s0 = jnp.sum(gat0 * gwr, axis=1, keepdims=True)
    sm = jnp.sum(Hm3 * gwr[:, None, :], axis=2)
    mx2 = jnp.maximum(jnp.max(sm, axis=1, keepdims=True), s0)
    e0 = jnp.exp(s0 - mx2)
    em = jnp.exp(sm - mx2)
    den2 = e0 + jnp.sum(em, axis=1, keepdims=True)
    pooled = (e0 / den2) * gat0 + jnp.sum(
        (em / den2)[:, :, None] * Hm3, axis=1)

    f1r = lax.dot_general(ob, fc1_ref[...], (((1,), (0,)), ((), ())),
                          preferred_element_type=jnp.float32)
    f1i = lax.dot_general(ob, fs1_ref[...], (((1,), (0,)), ((), ())),
                          preferred_element_type=jnp.float32)
    f2r = lax.dot_general(pooled, fc2_ref[...], (((1,), (0,)), ((), ())),
                          preferred_element_type=jnp.float32)
    f2i = lax.dot_general(pooled, fs2_ref[...], (((1,), (0,)), ((), ())),
                          preferred_element_type=jnp.float32)
    pr = f1r * f2r - f1i * f2i
    pi = f1r * f2i + f1i * f2r
    cb = (lax.dot_general(pr, ic_ref[...], (((1,), (0,)), ((), ())),
                          preferred_element_type=jnp.float32)
          - lax.dot_general(pi, isn_ref[...], (((1,), (0,)), ((), ())),
                            preferred_element_type=jnp.float32))

    o_ref[:, :VD] = cb
    o_ref[:, VD:] = ob


def _stage_d(out, sel3, Wg, A1, A2, E4, gwr, FC1, FS1, FC2, FS2, IC, ISn):
    full = lambda shape: pl.BlockSpec(shape, lambda i: tuple(0 for _ in shape))
    return pl.pallas_call(
        _compose_body,
        grid=(B // BBLK,),
        in_specs=[
            pl.BlockSpec((BBLK, VD), lambda i: (i, 0)),
            pl.BlockSpec((BBLK, RK, VD), lambda i: (i, 0, 0)),
            full((VD, VD)),
            full((VD, 4)),
            full((VD, 4)),
            full((4, VD)),
            full((1, VD)),
            full((VD, VD)),
            full((VD, VD)),
            full((VD, VD)),
            full((VD, VD)),
            full((VD, VD)),
            full((VD, VD)),
        ],
        out_specs=pl.BlockSpec((BBLK, 2 * VD), lambda i: (i, 0)),
        out_shape=jax.ShapeDtypeStruct((B, 2 * VD), jnp.float32),
    )(out, sel3, Wg, A1, A2, E4, gwr, FC1, FS1, FC2, FS2, IC, ISn)


_n = np.arange(128)
_ang = 2.0 * np.pi * ((np.outer(_n, _n) % 128).astype(np.float64)) / 128.0
_COS = np.cos(_ang)
_SIN = np.sin(_ang)


def kernel(out, key, idx, key_store, value_store, idx_store, Wp, bp, Wg,
           a_att, gat_attW, gat_attb, h1, h2, s1, s2):
    ksp = jnp.pad(key_store, ((0, SP - SLOTS), (0, 0)))
    bp2 = bp.reshape(1, KD)
    dh = VD // 4
    rows = jnp.arange(VD)
    head = rows // dh
    A1 = jnp.zeros((VD, 4), jnp.float32).at[rows, head].set(
        a_att[:, :dh].reshape(VD))
    A2 = jnp.zeros((VD, 4), jnp.float32).at[rows, head].set(
        a_att[:, dh:].reshape(VD))
    E4 = (head[None, :] == jnp.arange(4)[:, None]).astype(jnp.float32)
    gwr = gat_attW.reshape(1, VD)

    cosc = jnp.asarray(_COS, jnp.float32)
    sinc = jnp.asarray(_SIN, jnp.float32)
    FC1 = s1[:, None] * cosc[h1, :]
    FS1 = s1[:, None] * (-sinc)[h1, :]
    FC2 = s2[:, None] * cosc[h2, :]
    FS2 = s2[:, None] * (-sinc)[h2, :]
    IC = jnp.asarray(_COS / 128.0, jnp.float32)
    ISn = jnp.asarray(_SIN / 128.0, jnp.float32)

    sim3, M3 = _stage_a(key, Wp, bp2, ksp)
    M = M3.transpose(1, 0, 2).reshape(B, NCH)
    cids = _stage_b(M)
    crow = (cids * B + jnp.arange(B, dtype=jnp.int32)[:, None]).reshape(-1)
    cand = _gather_rows(sim3.reshape(NCH * B, C), crow)
    fids = _stage_c(cand.reshape(B, RK, C), cids)
    sel = _gather_rows(value_store, fids.reshape(-1))

    return _stage_d(out, sel.reshape(B, RK, VD), Wg, A1, A2, E4, gwr,
                    FC1, FS1, FC2, FS2, IC, ISn)


# speedup vs baseline: 10.6507x; 1.0825x over previous
import functools
import numpy as np
import jax
import jax.numpy as jnp
from jax import lax
from jax.experimental import pallas as pl
from jax.experimental.pallas import tpu as pltpu
from jax.experimental.pallas import tpu_sc as plsc

SLOTS = 100000
B = 1024
KD = 128
VD = 128
RK = 16
C = 128
SP = 102400
NCH = SP // C
SBLK = 4096
NSTEPS = SP // SBLK
CH_STEP = SBLK // C
BBLK = 256
NEG = -1e30
NEG2 = -3e38
IBIG = 2 ** 30

NCORES = 2
NSUB = 16
NW = NCORES * NSUB


def _sim_body(key_ref, wp_ref, bp_ref, ks_ref, sim_ref, m_ref, pkn_ref):
    i = pl.program_id(0)

    @pl.when(i == 0)
    def _():
        k = key_ref[...]
        pk = lax.dot_general(k, wp_ref[...], (((1,), (1,)), ((), ())),
                             preferred_element_type=jnp.float32) + bp_ref[...]
        n = jnp.maximum(jnp.sqrt(jnp.sum(pk * pk, axis=1, keepdims=True)), 1e-8)
        pkn_ref[...] = pk / n

    ks = ks_ref[...]
    pm = lax.dot_general(ks, wp_ref[...], (((1,), (1,)), ((), ())),
                         preferred_element_type=jnp.float32) + bp_ref[...]
    n = jnp.maximum(jnp.sqrt(jnp.sum(pm * pm, axis=1, keepdims=True)), 1e-8)
    pmn = pm / n
    simb = lax.dot_general(pkn_ref[...], pmn, (((1,), (1,)), ((), ())),
                           preferred_element_type=jnp.float32)
    col = lax.broadcasted_iota(jnp.int32, (B, SBLK), 1)
    simb = jnp.where(col + i * SBLK < SLOTS, simb, NEG)
    parts = []
    for c in range(CH_STEP):
        blk = simb[:, c * C:(c + 1) * C]
        sim_ref[c] = blk
        parts.append(jnp.max(blk, axis=1, keepdims=True))
    m_ref[...] = jnp.concatenate(parts, axis=1)[None]


def _stage_a(qkey, Wp, bp2, ksp):
    return pl.pallas_call(
        _sim_body,
        grid=(NSTEPS,),
        in_specs=[
            pl.BlockSpec((B, KD), lambda i: (0, 0)),
            pl.BlockSpec((KD, KD), lambda i: (0, 0)),
            pl.BlockSpec((1, KD), lambda i: (0, 0)),
            pl.BlockSpec((SBLK, KD), lambda i: (i, 0)),
        ],
        out_specs=[
            pl.BlockSpec((CH_STEP, B, C), lambda i: (i, 0, 0)),
            pl.BlockSpec((1, B, CH_STEP), lambda i: (i, 0, 0)),
        ],
        out_shape=[
            jax.ShapeDtypeStruct((NCH, B, C), jnp.float32),
            jax.ShapeDtypeStruct((NSTEPS, B, CH_STEP), jnp.float32),
        ],
        scratch_shapes=[pltpu.VMEM((B, KD), jnp.float32)],
    )(qkey, Wp, bp2, ksp)


def _extract_body(m_ref, ids_ref):
    x = m_ref[...]
    col = lax.broadcasted_iota(jnp.int32, (BBLK, NCH), 1)
    outs = []
    for _ in range(RK):
        mx = jnp.max(x, axis=1, keepdims=True)
        ci = jnp.where(x >= mx, col, IBIG)
        c = jnp.min(ci, axis=1, keepdims=True)
        outs.append(c)
        x = jnp.where(col == c, NEG2, x)
    ids_ref[...] = jnp.concatenate(outs, axis=1)


def _stage_b(M):
    return pl.pallas_call(
        _extract_body,
        grid=(B // BBLK,),
        in_specs=[pl.BlockSpec((BBLK, NCH), lambda i: (i, 0))],
        out_specs=pl.BlockSpec((BBLK, RK), lambda i: (i, 0)),
        out_shape=jax.ShapeDtypeStruct((B, RK), jnp.int32),
    )(M)


def _gather_rows(table, idx):
    nb = idx.shape[0]
    bpw = nb // NW
    nstream = bpw // 128
    idx2 = idx.reshape(nb // 128, 128)
    mesh = plsc.VectorSubcoreMesh(core_axis_name="c", subcore_axis_name="s",
                                  num_cores=NCORES, num_subcores=NSUB)

    @functools.partial(
        pl.kernel, mesh=mesh,
        out_type=jax.ShapeDtypeStruct((nb, VD), jnp.float32),
        scratch_types=[
            pltpu.VMEM((nstream, 128), jnp.int32),
            pltpu.VMEM((bpw, VD), jnp.float32),
            pltpu.SemaphoreType.DMA,
        ],
    )
    def k(table_hbm, idx_hbm, out_hbm, idx_v, rows_v, sem):
        wid = lax.axis_index("s") * NCORES + lax.axis_index("c")
        pltpu.sync_copy(idx_hbm.at[pl.ds(wid * nstream, nstream)], idx_v)
        copies = []
        for j in range(nstream):
            copies.append(pltpu.async_copy(
                table_hbm.at[idx_v.at[j]],
                rows_v.at[pl.ds(j * 128, 128)], sem))
        for cp in copies:
            cp.wait()
        pltpu.sync_copy(rows_v, out_hbm.at[pl.ds(wid * bpw, bpw)])

    return k(table, idx2)


def _rescan_body(cand_ref, cid_ref, ids_ref):
    x = cand_ref[...]
    cid = cid_ref[...]
    o = lax.broadcasted_iota(jnp.int32, (BBLK, RK, C), 2)
    slotid = cid[:, :, None] * C + o
    outs = []
    for _ in range(RK):
        mx = jnp.max(jnp.max(x, axis=2, keepdims=True), axis=1, keepdims=True)
        ci = jnp.where(x >= mx, slotid, IBIG)
        s = jnp.min(jnp.min(ci, axis=2, keepdims=True), axis=1, keepdims=True)
        outs.append(s[:, :, 0])
        x = jnp.where(slotid == s, NEG2, x)
    ids_ref[...] = jnp.concatenate(outs, axis=1)


def _stage_c(cand3, cids):
    return pl.pallas_call(
        _rescan_body,
        grid=(B // BBLK,),
        in_specs=[
            pl.BlockSpec((BBLK, RK, C), lambda i: (i, 0, 0)),
            pl.BlockSpec((BBLK, RK), lambda i: (i, 0)),
        ],
        out_specs=pl.BlockSpec((BBLK, RK), lambda i: (i, 0)),
        out_shape=jax.ShapeDtypeStruct((B, RK), jnp.int32),
    )(cand3, cids)


def _leaky(x):
    return jnp.where(x >= 0, x, 0.2 * x)


def _compose_body(out_ref, sel_ref, wg_ref, a1_ref, a2_ref, e4_ref, gw_ref,
                  fc1_ref, fs1_ref, fc2_ref, fs2_ref, ic_ref, isn_ref, o_ref):
    ob = out_ref[...]
    sel = sel_ref[...]
    Wg = wg_ref[...]
    selF = sel.reshape(BBLK * RK, VD)
    H0 = lax.dot_general(ob, Wg, (((1,), (0,)), ((), ())),
                         preferred_element_type=jnp.float32)
    Hm = lax.dot_general(selF, Wg, (((1,), (0,)), ((), ())),
                         preferred_element_type=jnp.float32)

    a1 = a1_ref[...]
    a2 = a2_ref[...]
    es0 = lax.dot_general(H0, a1, (((1,), (0,)), ((), ())),
                          preferred_element_type=jnp.float32)
    ed0 = lax.dot_general(H0, a2, (((1,), (0,)), ((), ())),
                          preferred_element_type=jnp.float32)
    edm = lax.dot_general(Hm, a2, (((1,), (0,)), ((), ())),
                          preferred_element_type=jnp.float32)
    edm3 = edm.reshape(BBLK, RK, 4)

    z0 = _leaky(es0 + ed0)
    zm3 = _leaky(es0[:, None, :] + edm3)
    mxz = jnp.maximum(jnp.max(zm3, axis=1), z0)
    w0 = jnp.exp(z0 - mxz)
    wm3 = jnp.exp(zm3 - mxz[:, None, :])
    den = w0 + jnp.sum(wm3, axis=1)
    a0 = w0 / den
    amF = (wm3 / den[:, None, :]).reshape(BBLK * RK, 4)

    e4 = e4_ref[...]
    w0E = lax.dot_general(a0, e4, (((1,), (0,)), ((), ())),
                          preferred_element_type=jnp.float32)
    wmE = lax.dot_general(amF, e4, (((1,), (0,)), ((), ())),
                          preferred_element_type=jnp.float32)
    gat0 = w0E * H0 + jnp.sum((wmE * Hm).reshape(BBLK, RK, VD), axis=1)

    gwr = gw_ref[...]
    Hm3 = Hm.reshape(BBLK, RK, VD)
    s0 = jnp.sum(gat0 * gwr, axis=1, keepdims=True)
    sm = jnp.sum(Hm3 * gwr[:, None, :], axis=2)
    mx2 = jnp.maximum(jnp.max(sm, axis=1, keepdims=True), s0)
    e0 = jnp.exp(s0 - mx2)
    em = jnp.exp(sm - mx2)
    den2 = e0 + jnp.sum(em, axis=1, keepdims=True)
    pooled = (e0 / den2) * gat0 + jnp.sum(
        (em / den2)[:, :, None] * Hm3, axis=1)

    f1r = lax.dot_general(ob, fc1_ref[...], (((1,), (0,)), ((), ())),
                          preferred_element_type=jnp.float32)
    f1i = lax.dot_general(ob, fs1_ref[...], (((1,), (0,)), ((), ())),
                          preferred_element_type=jnp.float32)
    f2r = lax.dot_general(pooled, fc2_ref[...], (((1,), (0,)), ((), ())),
                          preferred_element_type=jnp.float32)
    f2i = lax.dot_general(pooled, fs2_ref[...], (((1,), (0,)), ((), ())),
                          preferred_element_type=jnp.float32)
    pr = f1r * f2r - f1i * f2i
    pi = f1r * f2i + f1i * f2r
    cb = (lax.dot_general(pr, ic_ref[...], (((1,), (0,)), ((), ())),
                          preferred_element_type=jnp.float32)
          - lax.dot_general(pi, isn_ref[...], (((1,), (0,)), ((), ())),
                            preferred_element_type=jnp.float32))

    o_ref[:, :VD] = cb
    o_ref[:, VD:] = ob


def _stage_d(out, sel3, Wg, A1, A2, E4, gwr, FC1, FS1, FC2, FS2, IC, ISn):
    full = lambda shape: pl.BlockSpec(shape, lambda i: tuple(0 for _ in shape))
    return pl.pallas_call(
        _compose_body,
        grid=(B // BBLK,),
        in_specs=[
            pl.BlockSpec((BBLK, VD), lambda i: (i, 0)),
            pl.BlockSpec((BBLK, RK, VD), lambda i: (i, 0, 0)),
            full((VD, VD)),
            full((VD, 4)),
            full((VD, 4)),
            full((4, VD)),
            full((1, VD)),
            full((VD, VD)),
            full((VD, VD)),
            full((VD, VD)),
            full((VD, VD)),
            full((VD, VD)),
            full((VD, VD)),
        ],
        out_specs=pl.BlockSpec((BBLK, 2 * VD), lambda i: (i, 0)),
        out_shape=jax.ShapeDtypeStruct((B, 2 * VD), jnp.float32),
    )(out, sel3, Wg, A1, A2, E4, gwr, FC1, FS1, FC2, FS2, IC, ISn)


_n = np.arange(128)
_ang = 2.0 * np.pi * ((np.outer(_n, _n) % 128).astype(np.float64)) / 128.0
_COS = np.cos(_ang)
_SIN = np.sin(_ang)


def kernel(out, key, idx, key_store, value_store, idx_store, Wp, bp, Wg,
           a_att, gat_attW, gat_attb, h1, h2, s1, s2):
    ksp = key_store
    bp2 = bp.reshape(1, KD)
    dh = VD // 4
    rows = jnp.arange(VD)
    head = rows // dh
    A1 = jnp.zeros((VD, 4), jnp.float32).at[rows, head].set(
        a_att[:, :dh].reshape(VD))
    A2 = jnp.zeros((VD, 4), jnp.float32).at[rows, head].set(
        a_att[:, dh:].reshape(VD))
    E4 = (head[None, :] == jnp.arange(4)[:, None]).astype(jnp.float32)
    gwr = gat_attW.reshape(1, VD)

    cosc = jnp.asarray(_COS, jnp.float32)
    sinc = jnp.asarray(_SIN, jnp.float32)
    FC1 = s1[:, None] * cosc[h1, :]
    FS1 = s1[:, None] * (-sinc)[h1, :]
    FC2 = s2[:, None] * cosc[h2, :]
    FS2 = s2[:, None] * (-sinc)[h2, :]
    IC = jnp.asarray(_COS / 128.0, jnp.float32)
    ISn = jnp.asarray(_SIN / 128.0, jnp.float32)

    sim3, M3 = _stage_a(key, Wp, bp2, ksp)
    M = M3.transpose(1, 0, 2).reshape(B, NCH)
    cids = _stage_b(M)
    crow = (cids * B + jnp.arange(B, dtype=jnp.int32)[:, None]).reshape(-1)
    cand = _gather_rows(sim3.reshape(NCH * B, C), crow)
    fids = _stage_c(cand.reshape(B, RK, C), cids)
    sel = _gather_rows(value_store, fids.reshape(-1))

    return _stage_d(out, sel.reshape(B, RK, VD), Wg, A1, A2, E4, gwr,
                    FC1, FS1, FC2, FS2, IC, ISn)
